# Initial kernel scaffold; baseline (speedup 1.0000x reference)
import jax
import jax.numpy as jnp
from jax.experimental import pallas as pl


def kernel(x, emb_table, pos_table):
    raise NotImplementedError("write your pallas kernel here")


# speedup vs baseline: 2.7801x; 2.7801x over previous
import functools

import jax
import jax.numpy as jnp
from jax import lax
from jax.experimental import pallas as pl
from jax.experimental.pallas import tpu as pltpu
from jax.experimental.pallas import tpu_sc as plsc

VOCAB = 1000000
D = 64
B = 4096
S = 200

NC = 2
NS = 16
NW = NC * NS

SEQ_PER_W = B // NW
CHUNK_SEQ = 2
CHUNK = CHUNK_SEQ * S
NCHUNK = SEQ_PER_W // CHUNK_SEQ
ROWS_PER_W = SEQ_PER_W * S
GSPLIT = 5
GROWS = CHUNK // GSPLIT


def _emb_kernel(x_hbm, emb_hbm, pos_hbm, out_hbm, idx_v, rows_v, pos_v, sem):
    wid = lax.axis_index("s") * NC + lax.axis_index("c")
    wbase = wid * ROWS_PER_W

    pltpu.sync_copy(pos_hbm.at[pl.ds(0, S)], pos_v)

    def chunk_body(g, carry):
        base = pl.multiple_of(wbase + g * CHUNK, CHUNK)
        pltpu.sync_copy(x_hbm.at[pl.ds(base, CHUNK)], idx_v)
        handles = [
            pltpu.async_copy(
                emb_hbm.at[idx_v.at[pl.ds(j * GROWS, GROWS)]],
                rows_v.at[pl.ds(j * GROWS, GROWS)],
                sem,
            )
            for j in range(GSPLIT)
        ]
        for h in handles:
            h.wait()

        def add_body(r, c):
            for sq in range(CHUNK_SEQ):
                row = sq * S + r
                for d in range(D // 16):
                    sl = pl.ds(d * 16, 16)
                    rows_v[row, sl] = rows_v[row, sl] + pos_v[r, sl]
            return c

        lax.fori_loop(0, S, add_body, 0)
        pltpu.sync_copy(rows_v, out_hbm.at[pl.ds(base, CHUNK)])
        return carry

    lax.fori_loop(0, NCHUNK, chunk_body, 0)


@jax.jit
def kernel(x, emb_table, pos_table):
    x_flat = x.reshape(-1).astype(jnp.int32)
    mesh = plsc.VectorSubcoreMesh(core_axis_name="c", subcore_axis_name="s")
    out = pl.kernel(
        _emb_kernel,
        mesh=mesh,
        out_type=jax.ShapeDtypeStruct((B * S, D), jnp.float32),
        scratch_types=[
            pltpu.VMEM((CHUNK,), jnp.int32),
            pltpu.VMEM((CHUNK, D), jnp.float32),
            pltpu.VMEM((S, D), jnp.float32),
            pltpu.SemaphoreType.DMA,
        ],
        compiler_params=pltpu.CompilerParams(use_tc_tiling_on_sc=False),
    )(x_flat, emb_table, pos_table)
    return out.reshape(B, S, D)


# speedup vs baseline: 3.1217x; 1.1229x over previous
import functools

import jax
import jax.numpy as jnp
from jax import lax
from jax.experimental import pallas as pl
from jax.experimental.pallas import tpu as pltpu
from jax.experimental.pallas import tpu_sc as plsc

VOCAB = 1000000
D = 64
B = 4096
S = 200

NC = 2
NS = 16
NW = NC * NS

SEQ_PER_W = B // NW
CHUNK_SEQ = 2
CHUNK = CHUNK_SEQ * S
NCHUNK = SEQ_PER_W // CHUNK_SEQ
ROWS_PER_W = SEQ_PER_W * S
GSPLIT = 5
GROWS = CHUNK // GSPLIT


def _emb_kernel(x_hbm, emb_hbm, pos_hbm, out_hbm,
                idx_v, rows_v, pos_v, isems, gsems, osems):
    wid = lax.axis_index("s") * NC + lax.axis_index("c")
    wbase = wid * ROWS_PER_W

    pltpu.sync_copy(pos_hbm.at[pl.ds(0, S)], pos_v)

    def chunk_base(g):
        return pl.multiple_of(wbase + g * CHUNK, CHUNK)

    def idx_copy(g, b):
        return pltpu.make_async_copy(
            x_hbm.at[pl.ds(chunk_base(g), CHUNK)], idx_v.at[b], isems[b])

    def gather(b, j):
        return pltpu.make_async_copy(
            emb_hbm.at[idx_v.at[b].at[pl.ds(j * GROWS, GROWS)]],
            rows_v.at[b].at[pl.ds(j * GROWS, GROWS)],
            gsems[b])

    def out_copy(g, b):
        return pltpu.make_async_copy(
            rows_v.at[b], out_hbm.at[pl.ds(chunk_base(g), CHUNK)], osems[b])

    def add_pos(b):
        def add_body(r, c):
            for sq in range(CHUNK_SEQ):
                row = sq * S + r
                for d in range(D // 16):
                    sl = pl.ds(d * 16, 16)
                    rows_v[b, row, sl] = rows_v[b, row, sl] + pos_v[r, sl]
            return c
        lax.fori_loop(0, S, add_body, 0)

    idx_copy(0, 0).start()
    idx_copy(0, 0).wait()
    for j in range(GSPLIT):
        gather(0, j).start()
    idx_copy(1, 1).start()

    def pair_body(step, carry):
        for b in range(2):
            g = step * 2 + b
            nb = 1 - b
            for j in range(GSPLIT):
                gather(b, j).wait()

            @pl.when(g + 2 < NCHUNK)
            def _():
                idx_copy(g + 2, b).start()

            @pl.when(g >= 1)
            def _():
                out_copy(g - 1, nb).wait()

            @pl.when(g + 1 < NCHUNK)
            def _():
                idx_copy(g + 1, nb).wait()
                for j in range(GSPLIT):
                    gather(nb, j).start()

            add_pos(b)
            out_copy(g, b).start()
        return carry

    lax.fori_loop(0, NCHUNK // 2, pair_body, 0)

    out_copy(NCHUNK - 1, 1).wait()


@jax.jit
def kernel(x, emb_table, pos_table):
    x_flat = x.reshape(-1).astype(jnp.int32)
    mesh = plsc.VectorSubcoreMesh(core_axis_name="c", subcore_axis_name="s")
    out = pl.kernel(
        _emb_kernel,
        mesh=mesh,
        out_type=jax.ShapeDtypeStruct((B * S, D), jnp.float32),
        scratch_types=[
            pltpu.VMEM((2, CHUNK), jnp.int32),
            pltpu.VMEM((2, CHUNK, D), jnp.float32),
            pltpu.VMEM((S, D), jnp.float32),
            [pltpu.SemaphoreType.DMA, pltpu.SemaphoreType.DMA],
            [pltpu.SemaphoreType.DMA, pltpu.SemaphoreType.DMA],
            [pltpu.SemaphoreType.DMA, pltpu.SemaphoreType.DMA],
        ],
        compiler_params=pltpu.CompilerParams(use_tc_tiling_on_sc=False),
    )(x_flat, emb_table, pos_table)
    return out.reshape(B, S, D)
